# Initial kernel scaffold; baseline (speedup 1.0000x reference)
import functools

import jax
import jax.numpy as jnp
from jax import lax
from jax.experimental import pallas as pl
from jax.experimental.pallas import tpu as pltpu
from jax.experimental.pallas import tpu_sc as plsc

B = 16384
F = 26
V = 100000
D = 32
C = 13
H1 = 128
H2 = 64
NCLS = 2

NW = 32
PER_W = (B * F) // NW
IDX_ROW = 128
ROWS_PER_W = PER_W // IDX_ROW
ROWS_PER_CHUNK = 4
NCHUNK = ROWS_PER_W // ROWS_PER_CHUNK
CHUNK = ROWS_PER_CHUNK * IDX_ROW


def _gather_body(table_hbm, idx_hbm, out_hbm, idx_v, rows_v, sem):
    c = lax.axis_index("c")
    s = lax.axis_index("s")
    wid = s * 2 + c
    pltpu.sync_copy(idx_hbm.at[pl.ds(wid * ROWS_PER_W, ROWS_PER_W)], idx_v)
    base = wid * PER_W

    def chunk_body(ci, _):
        r0 = ci * ROWS_PER_CHUNK
        for j in range(ROWS_PER_CHUNK):
            pltpu.async_copy(
                table_hbm.at[idx_v.at[r0 + j]],
                rows_v.at[pl.ds(j * IDX_ROW, IDX_ROW)],
                sem,
            )
        for j in range(ROWS_PER_CHUNK):
            pltpu.make_async_copy(
                table_hbm.at[idx_v.at[r0 + j]],
                rows_v.at[pl.ds(j * IDX_ROW, IDX_ROW)],
                sem,
            ).wait()
        pltpu.sync_copy(rows_v, out_hbm.at[pl.ds(base + ci * CHUNK, CHUNK)])
        return 0

    lax.fori_loop(0, NCHUNK, chunk_body, 0)


@jax.jit
def _sc_gather(table_flat, idx2d):
    mesh = plsc.VectorSubcoreMesh(core_axis_name="c", subcore_axis_name="s")
    return pl.kernel(
        _gather_body,
        out_type=jax.ShapeDtypeStruct((B * F, D), jnp.float32),
        mesh=mesh,
        scratch_types=[
            pltpu.VMEM((ROWS_PER_W, IDX_ROW), jnp.int32),
            pltpu.VMEM((CHUNK, D), jnp.float32),
            pltpu.SemaphoreType.DMA,
        ],
    )(table_flat, idx2d)


def _mlp_body(emb_ref, xc_ref, w1a_ref, w1b_ref, b1_ref, w2_ref, b2_ref,
              w3_ref, b3_ref, out_ref):
    h = jnp.dot(emb_ref[...], w1a_ref[...], preferred_element_type=jnp.float32)
    h = h + jnp.dot(xc_ref[...], w1b_ref[...],
                    preferred_element_type=jnp.float32)
    h = jnp.maximum(h + b1_ref[...], 0.0)
    h = jnp.dot(h, w2_ref[...], preferred_element_type=jnp.float32)
    h = jnp.maximum(h + b2_ref[...], 0.0)
    o = jnp.dot(h, w3_ref[...], preferred_element_type=jnp.float32)
    out_ref[...] = o + b3_ref[...]


BM = 1024


@jax.jit
def _mlp(emb, xc_pad, w1a, w1b, b1, w2p, b2p, w3p, b3p):
    grid = (B // BM,)
    return pl.pallas_call(
        _mlp_body,
        grid=grid,
        in_specs=[
            pl.BlockSpec((BM, F * D), lambda i: (i, 0)),
            pl.BlockSpec((BM, 128), lambda i: (i, 0)),
            pl.BlockSpec((F * D, H1), lambda i: (0, 0)),
            pl.BlockSpec((128, H1), lambda i: (0, 0)),
            pl.BlockSpec((1, H1), lambda i: (0, 0)),
            pl.BlockSpec((H1, 128), lambda i: (0, 0)),
            pl.BlockSpec((1, 128), lambda i: (0, 0)),
            pl.BlockSpec((128, 128), lambda i: (0, 0)),
            pl.BlockSpec((1, 128), lambda i: (0, 0)),
        ],
        out_specs=pl.BlockSpec((BM, 128), lambda i: (i, 0)),
        out_shape=jax.ShapeDtypeStruct((B, 128), jnp.float32),
    )(emb, xc_pad, w1a, w1b, b1, w2p, b2p, w3p, b3p)


def kernel(x_cat, x_cont, tables, W1, b1, W2, b2, W3, b3):
    table_flat = tables.reshape(F * V, D)
    flat_idx = (x_cat.astype(jnp.int32)
                + (jnp.arange(F, dtype=jnp.int32) * V)[None, :])
    idx2d = flat_idx.reshape((B * F) // IDX_ROW, IDX_ROW)

    emb = _sc_gather(table_flat, idx2d).reshape(B, F * D)

    xc_pad = jnp.pad(x_cont, ((0, 0), (0, 128 - C)))
    w1a = W1[:F * D]
    w1b = jnp.pad(W1[F * D:], ((0, 128 - C), (0, 0)))
    w2p = jnp.pad(W2, ((0, 0), (0, 128 - H2)))
    b2p = jnp.pad(b2, (0, 128 - H2)).reshape(1, 128)
    w3p = jnp.pad(W3, ((0, 128 - H2), (0, 128 - NCLS)))
    b3p = jnp.pad(b3, (0, 128 - NCLS)).reshape(1, 128)

    out = _mlp(emb, xc_pad, w1a, w1b, b1.reshape(1, H1), w2p, b2p, w3p, b3p)
    return out[:, :NCLS]


# speedup vs baseline: 7.9632x; 7.9632x over previous
import functools

import jax
import jax.numpy as jnp
from jax import lax
from jax.experimental import pallas as pl
from jax.experimental.pallas import tpu as pltpu
from jax.experimental.pallas import tpu_sc as plsc

B = 16384
F = 26
V = 100000
D = 32
C = 13
H1 = 128
H2 = 64
NCLS = 2

NW = 32
PER_W = (B * F) // NW
IDX_ROW = 128
ROWS_PER_W = PER_W // IDX_ROW
ROWS_PER_CHUNK = 4
NCHUNK = ROWS_PER_W // ROWS_PER_CHUNK
CHUNK = ROWS_PER_CHUNK * IDX_ROW


def _gather_body(table_hbm, idx_hbm, out_hbm, idx_v, rows_v, sem):
    c = lax.axis_index("c")
    s = lax.axis_index("s")
    wid = s * 2 + c
    pltpu.sync_copy(idx_hbm.at[pl.ds(wid * ROWS_PER_W, ROWS_PER_W)], idx_v)
    base = wid * PER_W

    def chunk_body(ci, _):
        r0 = ci * ROWS_PER_CHUNK
        for j in range(ROWS_PER_CHUNK):
            pltpu.async_copy(
                table_hbm.at[idx_v.at[r0 + j]],
                rows_v.at[pl.ds(j * IDX_ROW, IDX_ROW)],
                sem,
            )
        for j in range(ROWS_PER_CHUNK):
            pltpu.make_async_copy(
                table_hbm.at[idx_v.at[r0 + j]],
                rows_v.at[pl.ds(j * IDX_ROW, IDX_ROW)],
                sem,
            ).wait()
        pltpu.sync_copy(rows_v, out_hbm.at[pl.ds(base + ci * CHUNK, CHUNK)])
        return 0

    lax.fori_loop(0, NCHUNK, chunk_body, 0)


@jax.jit
def _sc_gather(table_flat, idx2d):
    mesh = plsc.VectorSubcoreMesh(core_axis_name="c", subcore_axis_name="s")
    return pl.kernel(
        _gather_body,
        out_type=jax.ShapeDtypeStruct((B * F, D), jnp.float32),
        mesh=mesh,
        scratch_types=[
            pltpu.VMEM((ROWS_PER_W, IDX_ROW), jnp.int32),
            pltpu.VMEM((CHUNK, D), jnp.float32),
            pltpu.SemaphoreType.DMA,
        ],
        compiler_params=pltpu.CompilerParams(use_tc_tiling_on_sc=False),
    )(table_flat, idx2d)


def _mlp_body(emb_ref, xc_ref, w1a_ref, w1b_ref, b1_ref, w2_ref, b2_ref,
              w3_ref, b3_ref, out_ref):
    h = jnp.dot(emb_ref[...], w1a_ref[...], preferred_element_type=jnp.float32)
    h = h + jnp.dot(xc_ref[...], w1b_ref[...],
                    preferred_element_type=jnp.float32)
    h = jnp.maximum(h + b1_ref[...], 0.0)
    h = jnp.dot(h, w2_ref[...], preferred_element_type=jnp.float32)
    h = jnp.maximum(h + b2_ref[...], 0.0)
    o = jnp.dot(h, w3_ref[...], preferred_element_type=jnp.float32)
    out_ref[...] = o + b3_ref[...]


BM = 1024


@jax.jit
def _mlp(emb, xc_pad, w1a, w1b, b1, w2p, b2p, w3p, b3p):
    grid = (B // BM,)
    return pl.pallas_call(
        _mlp_body,
        grid=grid,
        in_specs=[
            pl.BlockSpec((BM, F * D), lambda i: (i, 0)),
            pl.BlockSpec((BM, 128), lambda i: (i, 0)),
            pl.BlockSpec((F * D, H1), lambda i: (0, 0)),
            pl.BlockSpec((128, H1), lambda i: (0, 0)),
            pl.BlockSpec((1, H1), lambda i: (0, 0)),
            pl.BlockSpec((H1, 128), lambda i: (0, 0)),
            pl.BlockSpec((1, 128), lambda i: (0, 0)),
            pl.BlockSpec((128, 128), lambda i: (0, 0)),
            pl.BlockSpec((1, 128), lambda i: (0, 0)),
        ],
        out_specs=pl.BlockSpec((BM, 128), lambda i: (i, 0)),
        out_shape=jax.ShapeDtypeStruct((B, 128), jnp.float32),
    )(emb, xc_pad, w1a, w1b, b1, w2p, b2p, w3p, b3p)


def kernel(x_cat, x_cont, tables, W1, b1, W2, b2, W3, b3):
    table_flat = tables.reshape(F * V, D)
    flat_idx = (x_cat.astype(jnp.int32)
                + (jnp.arange(F, dtype=jnp.int32) * V)[None, :])
    idx2d = flat_idx.reshape((B * F) // IDX_ROW, IDX_ROW)

    emb = _sc_gather(table_flat, idx2d).reshape(B, F * D)

    xc_pad = jnp.pad(x_cont, ((0, 0), (0, 128 - C)))
    w1a = W1[:F * D]
    w1b = jnp.pad(W1[F * D:], ((0, 128 - C), (0, 0)))
    w2p = jnp.pad(W2, ((0, 0), (0, 128 - H2)))
    b2p = jnp.pad(b2, (0, 128 - H2)).reshape(1, 128)
    w3p = jnp.pad(W3, ((0, 128 - H2), (0, 128 - NCLS)))
    b3p = jnp.pad(b3, (0, 128 - NCLS)).reshape(1, 128)

    out = _mlp(emb, xc_pad, w1a, w1b, b1.reshape(1, H1), w2p, b2p, w3p, b3p)
    return out[:, :NCLS]
